# baseline (speedup 1.0000x reference)
import functools

import jax
import jax.numpy as jnp
from jax import lax
from jax.experimental import pallas as pl
from jax.experimental.pallas import tpu as pltpu
from jax.experimental.pallas import tpu_sc as plsc

VOCAB = 1000000
DIM = 64
BATCH = 4096
SEQ = 200

_INFO = plsc.get_sparse_core_info()
NC = _INFO.num_cores
NS = _INFO.num_subcores

D_PER_C = DIM // NC
UNITS = -(-SEQ // NS)
PAIRS = UNITS // 2

CHUNK = 3072
N_FULL = VOCAB // CHUNK
TAIL = VOCAB - N_FULL * CHUNK
CPT = -(-N_FULL // NS)

_mesh = plsc.VectorSubcoreMesh(core_axis_name="c", subcore_axis_name="s")


@functools.partial(
    pl.kernel,
    mesh=_mesh,
    out_type=jax.ShapeDtypeStruct((SEQ, DIM, BATCH), jnp.float32),
    scratch_types=[
        pltpu.VMEM_SHARED((VOCAB,), jnp.float32),
        [pltpu.VMEM((CHUNK,), jnp.float32) for _ in range(2)],
        pltpu.VMEM((UNITS * BATCH,), jnp.int32),
        [pltpu.VMEM((BATCH,), jnp.float32) for _ in range(2)],
        pltpu.SemaphoreType.DMA,
        [pltpu.SemaphoreType.DMA for _ in range(2)],
        [pltpu.SemaphoreType.DMA for _ in range(2)],
        [pltpu.SemaphoreType.DMA for _ in range(2)],
    ],
    compiler_params=pltpu.CompilerParams(use_tc_tiling_on_sc=True),
)
def _gather_kernel(
    idx_f, tab_f, out_t, line, cbuf, idx_all, gbuf, l1sem, l2sem, gsem, ssem
):
    cid = lax.axis_index("c")
    sid = lax.axis_index("s")

    @pl.loop(0, UNITS)
    def _stage(k):
        s = sid + k * NS

        @pl.when(s < SEQ)
        def _():
            pltpu.sync_copy(
                idx_f.at[pl.ds(pl.multiple_of(s * BATCH, 8), BATCH)],
                idx_all.at[pl.ds(pl.multiple_of(k * BATCH, 8), BATCH)],
            )

    d0 = cid * D_PER_C

    def idx_of(k):
        return idx_all.at[pl.ds(pl.multiple_of(k * BATCH, 8), BATCH)]

    def drain_store(b):
        pltpu.make_async_copy(gbuf[b], out_t.at[0, 0, :], ssem[b]).wait()

    @pl.loop(0, D_PER_C)
    def _d_loop(i):
        d = d0 + i

        base = d * VOCAB

        @pl.loop(0, (CPT + 1) // 2)
        def _chunks(kk):
            for b in range(2):
                k = kk * 2 + b
                j = sid + k * NS

                @pl.when(j < N_FULL)
                def _(b=b, k=k, j=j):
                    off = pl.multiple_of(j * CHUNK, 8)

                    @pl.when(kk > 0)
                    def _():
                        pltpu.make_async_copy(
                            cbuf[b], line.at[pl.ds(0, CHUNK)], l2sem[b]
                        ).wait()

                    pltpu.async_copy(
                        tab_f.at[pl.ds(base + off, CHUNK)], cbuf[b], l1sem
                    ).wait()
                    pltpu.async_copy(cbuf[b], line.at[pl.ds(off, CHUNK)], l2sem[b])

        for b in range(2):
            pltpu.make_async_copy(cbuf[b], line.at[pl.ds(0, CHUNK)], l2sem[b]).wait()

        @pl.when(sid == NS - 1)
        def _():
            toff = pl.multiple_of(N_FULL * CHUNK, 8)
            pltpu.sync_copy(
                tab_f.at[pl.ds(base + toff, TAIL)], cbuf[0].at[pl.ds(0, TAIL)]
            )
            pltpu.sync_copy(cbuf[0].at[pl.ds(0, TAIL)], line.at[pl.ds(toff, TAIL)])

        plsc.subcore_barrier()

        def unit_start(u, b, first_use):
            s = sid + u * NS

            @pl.when(s < SEQ)
            def _():
                @pl.when(jnp.logical_not(first_use))
                def _():
                    drain_store(b)

                pltpu.async_copy(line.at[idx_of(u)], gbuf[b], gsem[b])

            return s

        def unit_finish(u, b, s):
            @pl.when(s < SEQ)
            def _():
                pltpu.make_async_copy(line.at[idx_of(u)], gbuf[b], gsem[b]).wait()
                pltpu.async_copy(gbuf[b], out_t.at[s, d, :], ssem[b])

        @pl.loop(0, PAIRS)
        def _pairs(r):
            u0 = r * 2
            u1 = u0 + 1
            fresh = jnp.logical_and(i == 0, r == 0)
            s0 = unit_start(u0, 0, fresh)
            s1 = unit_start(u1, 1, fresh)
            unit_finish(u0, 0, s0)
            unit_finish(u1, 1, s1)

        s12 = unit_start(UNITS - 1, 0, jnp.bool_(False))
        unit_finish(UNITS - 1, 0, s12)

        plsc.subcore_barrier()

    for b in range(2):
        drain_store(b)


def kernel(input, table):
    idx_f = input.T.reshape(SEQ * BATCH)
    tab_f = table.T.reshape(DIM * VOCAB)
    out_t = _gather_kernel(idx_f, tab_f)
    return out_t.transpose(2, 0, 1)


# speedup vs baseline: 8.6136x; 8.6136x over previous
import functools

import jax
import jax.numpy as jnp
from jax import lax
from jax.experimental import pallas as pl
from jax.experimental.pallas import tpu as pltpu
from jax.experimental.pallas import tpu_sc as plsc

VOCAB = 1000000
DIM = 64
BATCH = 4096
SEQ = 200

_INFO = plsc.get_sparse_core_info()
NC = _INFO.num_cores
NS = _INFO.num_subcores

D_PER_C = DIM // NC
UNITS = -(-SEQ // NS)
NSL = 3
ROUNDS = (UNITS - 1) // NSL

VMAIN = (VOCAB // 128) * 128
VTAIL = VOCAB - VMAIN
STRIP = 62464
REM = VMAIN - NS * STRIP
LINE = VMAIN + 128

_mesh = plsc.VectorSubcoreMesh(core_axis_name="c", subcore_axis_name="s")


@functools.partial(
    pl.kernel,
    mesh=_mesh,
    out_type=jax.ShapeDtypeStruct((SEQ, DIM, BATCH), jnp.float32),
    scratch_types=[
        pltpu.VMEM_SHARED((LINE,), jnp.float32),
        pltpu.VMEM((UNITS * BATCH,), jnp.int32),
        [pltpu.VMEM((BATCH,), jnp.float32) for _ in range(NSL)],
        pltpu.SemaphoreType.DMA,
        [pltpu.SemaphoreType.DMA for _ in range(NSL)],
        [pltpu.SemaphoreType.DMA for _ in range(NSL)],
    ],
    compiler_params=pltpu.CompilerParams(use_tc_tiling_on_sc=True),
)
def _gather_kernel(idx3, tab3, tail3, out_t, line, idx_all, gbuf, lsem, gsem, ssem):
    cid = lax.axis_index("c")
    sid = lax.axis_index("s")

    @pl.loop(0, UNITS)
    def _stage(k):
        s = sid + k * NS

        @pl.when(s < SEQ)
        def _():
            pltpu.sync_copy(
                idx3.at[s // 8, s % 8, :],
                idx_all.at[pl.ds(pl.multiple_of(k * BATCH, 8), BATCH)],
            )

    d0 = cid * D_PER_C

    def idx_of(k):
        return idx_all.at[pl.ds(pl.multiple_of(k * BATCH, 8), BATCH)]

    def drain_store(b):
        pltpu.make_async_copy(gbuf[b], out_t.at[0, 0, :], ssem[b]).wait()

    @pl.loop(0, D_PER_C)
    def _d_loop(i):
        d = d0 + i
        dR = d // 8
        dr = d % 8

        off = pl.multiple_of(sid * STRIP, 128)
        pltpu.async_copy(
            tab3.at[dR, dr, pl.ds(off, STRIP)], line.at[pl.ds(off, STRIP)], lsem
        )

        @pl.when(sid == 0)
        def _():
            roff = pl.multiple_of(NS * STRIP, 128)
            pltpu.async_copy(
                tab3.at[dR, dr, pl.ds(roff, REM)], line.at[pl.ds(roff, REM)], lsem
            )
            pltpu.async_copy(
                tail3.at[dR, dr, :], line.at[pl.ds(VMAIN, 128)], lsem
            )
            pltpu.make_async_copy(
                tab3.at[dR, dr, pl.ds(roff, REM)], line.at[pl.ds(roff, REM)], lsem
            ).wait()
            pltpu.make_async_copy(
                tail3.at[dR, dr, :], line.at[pl.ds(VMAIN, 128)], lsem
            ).wait()

        pltpu.make_async_copy(
            tab3.at[dR, dr, pl.ds(off, STRIP)], line.at[pl.ds(off, STRIP)], lsem
        ).wait()
        plsc.subcore_barrier()

        def unit_start(u, b, first_use):
            s = sid + u * NS

            @pl.when(s < SEQ)
            def _():
                @pl.when(jnp.logical_not(first_use))
                def _():
                    drain_store(b)

                pltpu.async_copy(line.at[idx_of(u)], gbuf[b], gsem[b])

            return s

        def unit_finish(u, b, s):
            @pl.when(s < SEQ)
            def _():
                pltpu.make_async_copy(line.at[idx_of(u)], gbuf[b], gsem[b]).wait()
                pltpu.async_copy(gbuf[b], out_t.at[s, d, :], ssem[b])

        @pl.loop(0, ROUNDS)
        def _rounds(r):
            fresh = jnp.logical_and(i == 0, r == 0)
            ss = [unit_start(r * NSL + b, b, fresh) for b in range(NSL)]
            for b in range(NSL):
                unit_finish(r * NSL + b, b, ss[b])

        s12 = unit_start(UNITS - 1, 0, jnp.bool_(False))
        unit_finish(UNITS - 1, 0, s12)

        plsc.subcore_barrier()

    for b in range(NSL):
        drain_store(b)


def kernel(input, table):
    tab_t = table.T
    idx3 = input.T.reshape(SEQ // 8, 8, BATCH)
    tab3 = tab_t.reshape(8, DIM // 8, VOCAB)
    tail3 = jnp.pad(tab_t[:, VMAIN:], ((0, 0), (0, 128 - VTAIL))).reshape(
        8, DIM // 8, 128
    )
    out_t = _gather_kernel(idx3, tab3, tail3)
    return out_t.transpose(2, 0, 1)
